# Initial kernel scaffold; baseline (speedup 1.0000x reference)
import jax
import jax.numpy as jnp
from jax.experimental import pallas as pl


def kernel(feats, context, proj_matrix, codebook, W_enc, b_enc):
    raise NotImplementedError("write your pallas kernel here")


# speedup vs baseline: 9.0687x; 9.0687x over previous
import functools

import jax
import jax.numpy as jnp
from jax.experimental import pallas as pl

_N, _T, _FEAT, _VEC, _K, _DIN = 4, 1024, 512, 32, 512, 512
_ROWS = _N * _T


def _loss_kernel(feats_ref, ctx_ref, proj_ref, cbt_ref, w_ref, b_ref, out_ref,
                 *, block_rows, n_rows, n_classes):
    step = pl.program_id(0)

    g = jnp.dot(feats_ref[...], proj_ref[...],
                preferred_element_type=jnp.float32)
    scores = jnp.dot(g, cbt_ref[...],
                     preferred_element_type=jnp.float32)
    logits = jnp.dot(ctx_ref[...], w_ref[...],
                     preferred_element_type=jnp.float32) + b_ref[...]

    m = jnp.max(logits, axis=1, keepdims=True)
    lse = m[:, 0] + jnp.log(jnp.sum(jnp.exp(logits - m), axis=1))

    smax = jnp.max(scores, axis=1, keepdims=True)
    iota = jax.lax.broadcasted_iota(jnp.int32, scores.shape, 1)
    t = jnp.min(jnp.where(scores == smax, iota, n_classes), axis=1,
                keepdims=True)
    picked = jnp.sum(jnp.where(iota == t, logits, 0.0), axis=1)

    partial = (jnp.sum(lse - picked) * (1.0 / n_rows)).reshape(1, 1)

    @pl.when(step == 0)
    def _init():
        out_ref[...] = jnp.zeros_like(out_ref)

    out_ref[...] += partial


def kernel(feats, context, proj_matrix, codebook, W_enc, b_enc):
    feats2 = feats.reshape(_ROWS, _FEAT)
    ctx2 = context.reshape(_ROWS, _DIN)
    cbt = codebook.T
    b2 = b_enc.reshape(1, _K)

    block_rows = 512
    grid = (_ROWS // block_rows,)

    out = pl.pallas_call(
        functools.partial(_loss_kernel, block_rows=block_rows,
                          n_rows=_ROWS, n_classes=_K),
        grid=grid,
        in_specs=[
            pl.BlockSpec((block_rows, _FEAT), lambda i: (i, 0)),
            pl.BlockSpec((block_rows, _DIN), lambda i: (i, 0)),
            pl.BlockSpec((_FEAT, _VEC), lambda i: (0, 0)),
            pl.BlockSpec((_VEC, _K), lambda i: (0, 0)),
            pl.BlockSpec((_DIN, _K), lambda i: (0, 0)),
            pl.BlockSpec((1, _K), lambda i: (0, 0)),
        ],
        out_specs=pl.BlockSpec((1, 1), lambda i: (0, 0)),
        out_shape=jax.ShapeDtypeStruct((1, 1), jnp.float32),
    )(feats2, ctx2, proj_matrix, cbt, W_enc, b2)

    return out[0, 0]
